# Initial kernel scaffold; baseline (speedup 1.0000x reference)
import jax
import jax.numpy as jnp
from jax.experimental import pallas as pl


def kernel(x, W1, b1, W2, b2):
    raise NotImplementedError("write your pallas kernel here")


# speedup vs baseline: 11.5787x; 11.5787x over previous
import functools

import jax
import jax.numpy as jnp
from jax import lax
from jax.experimental import pallas as pl
from jax.experimental.pallas import tpu as pltpu
from jax.experimental.pallas import tpu_sc as plsc

B = 4096
D = 32
H = 32
K = 5
BLK = 512
NC, NS = 2, 16
NW = NC * NS
RPW = B // NW


def _tc_body(xb_ref, xf_ref, w1_ref, b1_ref, w2_ref, b2_ref, idx_ref, y_ref):
    xb = xb_ref[...]
    xf = xf_ref[...]

    h = lax.dot_general(xb, w1_ref[...], (((1,), (1,)), ((), ())),
                        preferred_element_type=jnp.float32)
    h = jnp.maximum(h + b1_ref[...], 0.0)
    y = lax.dot_general(h, w2_ref[...], (((1,), (1,)), ((), ())),
                        preferred_element_type=jnp.float32)
    y_ref[...] = y * 0.2 + b2_ref[...] * 0.2

    sqb = jnp.sum(xb * xb, axis=1)
    sqf = jnp.sum(xf * xf, axis=1)
    g = lax.dot_general(xb, xf, (((1,), (1,)), ((), ())),
                        preferred_element_type=jnp.float32)
    neg = 2.0 * g - sqb[:, None] - sqf[None, :]
    negc = jnp.minimum(neg, 0.0)

    cols = lax.broadcasted_iota(jnp.int32, (BLK, B), 1)
    for j in range(K):
        m = jnp.max(negc, axis=1)
        eq = negc == m[:, None]
        sel = jnp.min(jnp.where(eq, cols, B), axis=1)
        idx_ref[j, :] = sel
        negc = jnp.where(cols == sel[:, None], -jnp.inf, negc)


def _tc_knn(x, w1, b1, w2, b2):
    grid = B // BLK
    return pl.pallas_call(
        _tc_body,
        grid=(grid,),
        in_specs=[
            pl.BlockSpec((BLK, D), lambda i: (i, 0)),
            pl.BlockSpec((B, D), lambda i: (0, 0)),
            pl.BlockSpec((H, D), lambda i: (0, 0)),
            pl.BlockSpec((1, H), lambda i: (0, 0)),
            pl.BlockSpec((H, H), lambda i: (0, 0)),
            pl.BlockSpec((1, H), lambda i: (0, 0)),
        ],
        out_specs=[
            pl.BlockSpec((K, BLK), lambda i: (0, i)),
            pl.BlockSpec((BLK, H), lambda i: (i, 0)),
        ],
        out_shape=[
            jax.ShapeDtypeStruct((K, B), jnp.int32),
            jax.ShapeDtypeStruct((B, H), jnp.float32),
        ],
    )(x, x, w1, b1.reshape(1, H), w2, b2.reshape(1, H))


@functools.cache
def _build_sc_gather_sum():
    mesh = plsc.VectorSubcoreMesh(core_axis_name="c", subcore_axis_name="s")

    @functools.partial(
        pl.kernel,
        mesh=mesh,
        out_type=jax.ShapeDtypeStruct((B, H), jnp.float32),
        scratch_types=[
            pltpu.VMEM((K, RPW), jnp.int32),
            pltpu.VMEM((K, RPW, H), jnp.float32),
            pltpu.VMEM((RPW, H), jnp.float32),
            pltpu.SemaphoreType.DMA,
        ],
        compiler_params=pltpu.CompilerParams(use_tc_tiling_on_sc=False),
    )
    def _sc_gather_sum(y_hbm, idx_hbm, out_hbm, idx_v, rows_v, out_v, sem):
        wid = lax.axis_index("s") * NC + lax.axis_index("c")
        base = wid * RPW
        for j in range(K):
            pltpu.sync_copy(idx_hbm.at[pl.ds(j * B + base, RPW)], idx_v.at[j])
        copies = [
            pltpu.async_copy(y_hbm.at[idx_v.at[j]], rows_v.at[j], sem)
            for j in range(K)
        ]
        for c in copies:
            c.wait()

        def body(r, carry):
            for c in (0, 16):
                s = rows_v[0, r, pl.ds(c, 16)]
                for j in range(1, K):
                    s = s + rows_v[j, r, pl.ds(c, 16)]
                out_v[r, pl.ds(c, 16)] = s
            return carry

        lax.fori_loop(0, RPW, body, 0)
        pltpu.sync_copy(out_v, out_hbm.at[pl.ds(base, RPW)])

    return _sc_gather_sum


def kernel(x, W1, b1, W2, b2):
    idx, y = _tc_knn(x, W1, b1, W2, b2)
    return _build_sc_gather_sum()(y, idx.reshape(K * B))


# speedup vs baseline: 14.9252x; 1.2890x over previous
import functools

import jax
import jax.numpy as jnp
from jax import lax
from jax.experimental import pallas as pl
from jax.experimental.pallas import tpu as pltpu
from jax.experimental.pallas import tpu_sc as plsc

B = 4096
D = 32
H = 32
K = 5
BLK = 1024
NC, NS = 2, 16
NW = NC * NS
RPW = B // NW
WPB = BLK // RPW


def _tc_body(xb_ref, xf_ref, w1_ref, b1_ref, w2_ref, b2_ref, idx_ref, y_ref):
    xb = xb_ref[...]
    xf = xf_ref[...]

    h = lax.dot_general(xb, w1_ref[...], (((1,), (1,)), ((), ())),
                        preferred_element_type=jnp.float32)
    h = jnp.maximum(h + b1_ref[...], 0.0)
    y = lax.dot_general(h, w2_ref[...], (((1,), (1,)), ((), ())),
                        preferred_element_type=jnp.float32)
    y_ref[...] = y * 0.2 + b2_ref[...] * 0.2

    sqb = jnp.sum(xb * xb, axis=1)
    sqf = jnp.sum(xf * xf, axis=1)
    g = lax.dot_general(xb, xf, (((1,), (1,)), ((), ())),
                        preferred_element_type=jnp.float32)
    d2 = jnp.maximum(sqb[:, None] + sqf[None, :] - 2.0 * g, 0.0)

    i = pl.program_id(0)

    def store(j, sel):
        for w in range(WPB):
            off = (i * WPB + w) * (K * RPW) + j * RPW
            idx_ref[pl.ds(off, RPW)] = sel[w * RPW:(w + 1) * RPW]

    cols = lax.broadcasted_iota(jnp.int32, (BLK, B), 1)
    rows = i * BLK + lax.iota(jnp.int32, BLK)
    store(0, rows)
    d2 = jnp.where(cols == rows[:, None], jnp.inf, d2)
    for j in range(1, K):
        sel = jnp.argmin(d2, axis=1).astype(jnp.int32)
        store(j, sel)
        d2 = jnp.where(cols == sel[:, None], jnp.inf, d2)


def _tc_knn(x, w1, b1, w2, b2):
    grid = B // BLK
    return pl.pallas_call(
        _tc_body,
        grid=(grid,),
        in_specs=[
            pl.BlockSpec((BLK, D), lambda i: (i, 0)),
            pl.BlockSpec((B, D), lambda i: (0, 0)),
            pl.BlockSpec((H, D), lambda i: (0, 0)),
            pl.BlockSpec((1, H), lambda i: (0, 0)),
            pl.BlockSpec((H, H), lambda i: (0, 0)),
            pl.BlockSpec((1, H), lambda i: (0, 0)),
        ],
        out_specs=[
            pl.BlockSpec((NW * K * RPW,), lambda i: (0,)),
            pl.BlockSpec((BLK, H), lambda i: (i, 0)),
        ],
        out_shape=[
            jax.ShapeDtypeStruct((NW * K * RPW,), jnp.int32),
            jax.ShapeDtypeStruct((B, H), jnp.float32),
        ],
    )(x, x, w1, b1.reshape(1, H), w2, b2.reshape(1, H))


@functools.cache
def _build_sc_gather_sum():
    mesh = plsc.VectorSubcoreMesh(core_axis_name="c", subcore_axis_name="s")

    @functools.partial(
        pl.kernel,
        mesh=mesh,
        out_type=jax.ShapeDtypeStruct((B, H), jnp.float32),
        scratch_types=[
            pltpu.VMEM((K * RPW,), jnp.int32),
            pltpu.VMEM((K, RPW, H), jnp.float32),
            pltpu.VMEM((RPW, H), jnp.float32),
            pltpu.SemaphoreType.DMA,
        ],
        compiler_params=pltpu.CompilerParams(use_tc_tiling_on_sc=False),
    )
    def _sc_gather_sum(y_hbm, idx_hbm, out_hbm, idx_v, rows_v, out_v, sem):
        wid = lax.axis_index("s") * NC + lax.axis_index("c")
        base = wid * RPW
        pltpu.sync_copy(idx_hbm.at[pl.ds(wid * (K * RPW), K * RPW)], idx_v)
        copies = [
            pltpu.async_copy(y_hbm.at[idx_v.at[pl.ds(j * RPW, RPW)]],
                             rows_v.at[j], sem)
            for j in range(K)
        ]
        for c in copies:
            c.wait()

        def body(r, carry):
            for c in (0, 16):
                s = rows_v[0, r, pl.ds(c, 16)]
                for j in range(1, K):
                    s = s + rows_v[j, r, pl.ds(c, 16)]
                out_v[r, pl.ds(c, 16)] = s
            return carry

        lax.fori_loop(0, RPW, body, 0)
        pltpu.sync_copy(out_v, out_hbm.at[pl.ds(base, RPW)])

    return _sc_gather_sum


def kernel(x, W1, b1, W2, b2):
    idx, y = _tc_knn(x, W1, b1, W2, b2)
    return _build_sc_gather_sum()(y, idx)
